# baseline (speedup 1.0000x reference)
import functools

import jax
import jax.numpy as jnp
from jax import lax
from jax.experimental import pallas as pl
from jax.experimental.pallas import tpu as pltpu
from jax.experimental.pallas import tpu_sc as plsc

B = 16384
K = 32
NC = 2
NS = 16
NW = NC * NS
BPW = B // NW
CHUNK = 128
NCHUNK = BPW // CHUNK

RATING_RANGE = 4.5
LOWEST_RATING = 0.5

_mesh = plsc.VectorSubcoreMesh(core_axis_name="c", subcore_axis_name="s")


@functools.partial(
    pl.kernel,
    mesh=_mesh,
    compiler_params=pltpu.CompilerParams(use_tc_tiling_on_sc=False),
    out_type=[
        jax.ShapeDtypeStruct((B, K), jnp.float32),
        jax.ShapeDtypeStruct((B, K), jnp.float32),
    ],
    scratch_types=[
        pltpu.VMEM((NCHUNK, CHUNK), jnp.int32),
        pltpu.VMEM((NCHUNK, CHUNK), jnp.int32),
        pltpu.VMEM((BPW, K), jnp.float32),
        pltpu.VMEM((BPW, K), jnp.float32),
        pltpu.SemaphoreType.DMA,
        pltpu.SemaphoreType.DMA,
    ],
)
def _gather_sc(users_hbm, items_hbm, u_hbm, i_hbm, ux_hbm, ix_hbm,
               uidx_v, iidx_v, urows_v, irows_v, sem_u, sem_i):
    wid = lax.axis_index("s") * NC + lax.axis_index("c")
    base = wid * BPW
    pltpu.sync_copy(users_hbm.at[pl.ds(wid * NCHUNK, NCHUNK)], uidx_v)
    pltpu.sync_copy(items_hbm.at[pl.ds(wid * NCHUNK, NCHUNK)], iidx_v)
    copies = []
    for j in range(NCHUNK):
        copies.append(pltpu.async_copy(
            u_hbm.at[uidx_v.at[j]], urows_v.at[pl.ds(j * CHUNK, CHUNK)], sem_u))
        copies.append(pltpu.async_copy(
            i_hbm.at[iidx_v.at[j]], irows_v.at[pl.ds(j * CHUNK, CHUNK)], sem_i))
    for c in copies:
        c.wait()
    pltpu.sync_copy(urows_v, ux_hbm.at[pl.ds(base, BPW)])
    pltpu.sync_copy(irows_v, ix_hbm.at[pl.ds(base, BPW)])


def _mlp_body(ux_ref, ix_ref, w1u_ref, w1i_ref, b1_ref, w2_ref, b2_ref,
              woutt_ref, bout_ref, out_ref):
    h = jnp.dot(ux_ref[...], w1u_ref[...],
                preferred_element_type=jnp.float32,
                precision=lax.Precision.HIGHEST)
    h = h + jnp.dot(ix_ref[...], w1i_ref[...],
                    preferred_element_type=jnp.float32,
                    precision=lax.Precision.HIGHEST)
    h = jnp.maximum(h + b1_ref[...], 0.0)
    h = jnp.maximum(
        jnp.dot(h, w2_ref[...], preferred_element_type=jnp.float32,
                precision=lax.Precision.HIGHEST) + b2_ref[...], 0.0)
    o = jnp.sum(h * woutt_ref[...], axis=1, keepdims=True) + bout_ref[...]
    out_ref[...] = jax.nn.sigmoid(o) * RATING_RANGE + LOWEST_RATING


def kernel(users, items, U, I, W1, b1, W2, b2, Wout, bout):
    users2d = users.astype(jnp.int32).reshape(NW * NCHUNK, CHUNK)
    items2d = items.astype(jnp.int32).reshape(NW * NCHUNK, CHUNK)
    ux, ix = _gather_sc(users2d, items2d, U, I)

    w1u = W1[:K, :]
    w1i = W1[K:, :]
    bb = 4096
    full = lambda shape: pl.BlockSpec(shape, lambda i: (0, 0))
    out = pl.pallas_call(
        _mlp_body,
        grid=(B // bb,),
        in_specs=[
            pl.BlockSpec((bb, K), lambda i: (i, 0)),
            pl.BlockSpec((bb, K), lambda i: (i, 0)),
            full((K, 64)),
            full((K, 64)),
            full((1, 64)),
            full((64, 32)),
            full((1, 32)),
            full((1, 32)),
            full((1, 1)),
        ],
        out_specs=pl.BlockSpec((bb, 1), lambda i: (i, 0)),
        out_shape=jax.ShapeDtypeStruct((B, 1), jnp.float32),
    )(ux, ix, w1u, w1i, b1.reshape(1, -1), W2, b2.reshape(1, -1),
      Wout.reshape(1, -1), bout.reshape(1, 1))
    return out
